# baseline (speedup 1.0000x reference)
import jax
import jax.numpy as jnp
from jax.experimental import pallas as pl
from jax.experimental.pallas import tpu as pltpu

_BLOCK_ROWS = 512


def _binarize_kernel(x_ref, m_ref, o_ref):
    m = m_ref[...]
    o_ref[...] = jnp.logical_and(m > 0.0, x_ref[...] >= m)


def kernel(inputs, medians):
    n, f = inputs.shape
    m2 = medians.reshape(1, f)
    grid = (n // _BLOCK_ROWS,)
    return pl.pallas_call(
        _binarize_kernel,
        grid=grid,
        in_specs=[
            pl.BlockSpec((_BLOCK_ROWS, f), lambda i: (i, 0)),
            pl.BlockSpec((1, f), lambda i: (0, 0)),
        ],
        out_specs=pl.BlockSpec((_BLOCK_ROWS, f), lambda i: (i, 0)),
        out_shape=jax.ShapeDtypeStruct((n, f), jnp.bool_),
        compiler_params=pltpu.CompilerParams(
            dimension_semantics=("parallel",),
        ),
    )(inputs, m2)


# speedup vs baseline: 1.5230x; 1.5230x over previous
import jax
import jax.numpy as jnp
from jax.experimental import pallas as pl
from jax.experimental.pallas import tpu as pltpu

_BLOCK_ROWS = 512


def _binarize_kernel(x_ref, m_ref, o_ref):
    m = m_ref[...]
    o_ref[...] = jnp.logical_and(m > 0.0, x_ref[...] >= m).astype(jnp.int8)


def kernel(inputs, medians):
    n, f = inputs.shape
    m2 = medians.reshape(1, f)
    grid = (n // _BLOCK_ROWS,)
    out_i8 = pl.pallas_call(
        _binarize_kernel,
        grid=grid,
        in_specs=[
            pl.BlockSpec((_BLOCK_ROWS, f), lambda i: (i, 0)),
            pl.BlockSpec((1, f), lambda i: (0, 0)),
        ],
        out_specs=pl.BlockSpec((_BLOCK_ROWS, f), lambda i: (i, 0)),
        out_shape=jax.ShapeDtypeStruct((n, f), jnp.int8),
        compiler_params=pltpu.CompilerParams(
            dimension_semantics=("parallel",),
        ),
    )(inputs, m2)
    return out_i8.astype(jnp.bool_)
